# Initial kernel scaffold; baseline (speedup 1.0000x reference)
import jax
import jax.numpy as jnp
from jax.experimental import pallas as pl


def kernel(x_A, x_B, train_pos_edge_index, pos_edge_index, neg_edge_index, W1, b1, W2, b2, Wm, bm, Wa, ba):
    raise NotImplementedError("write your pallas kernel here")


# speedup vs baseline: 1.1866x; 1.1866x over previous
import functools

import jax
import jax.numpy as jnp
from jax.experimental import pallas as pl
from jax.experimental.pallas import tpu as pltpu

N = 10000
F_IN = 256
H = 128
O = 64
C = 16


def _mm_body(x_ref, w_ref, o_ref):
    o_ref[...] = jnp.dot(x_ref[...], w_ref[...],
                         preferred_element_type=jnp.float32)


def _mm(x, w, block_rows=2000):
    m, k = x.shape
    n = w.shape[1]
    grid = (m // block_rows,)
    return pl.pallas_call(
        _mm_body,
        grid=grid,
        in_specs=[
            pl.BlockSpec((block_rows, k), lambda i: (i, 0)),
            pl.BlockSpec((k, n), lambda i: (0, 0)),
        ],
        out_specs=pl.BlockSpec((block_rows, n), lambda i: (i, 0)),
        out_shape=jax.ShapeDtypeStruct((m, n), jnp.float32),
    )(x, w)


def _gcn_conv(x, edge_index, W, b, dinv):
    n = x.shape[0]
    h = _mm(x, W)
    src = edge_index[0]
    dst = edge_index[1]
    norm = dinv[src] * dinv[dst]
    msg = h[src] * norm[:, None]
    out = jax.ops.segment_sum(msg, dst, num_segments=n)
    out = out + h * (dinv * dinv)[:, None]
    return out + b


def kernel(x_A, x_B, train_pos_edge_index, pos_edge_index, neg_edge_index,
           W1, b1, W2, b2, Wm, bm, Wa, ba):
    n = x_A.shape[0]
    dst = train_pos_edge_index[1]
    deg = jax.ops.segment_sum(jnp.ones(dst.shape[0], jnp.float32), dst,
                              num_segments=n) + 1.0
    dinv = jax.lax.rsqrt(deg)

    x1 = jax.nn.relu(_gcn_conv(x_A, train_pos_edge_index, W1, b1, dinv))
    x1 = _gcn_conv(x1, train_pos_edge_index, W2, b2, dinv)
    x2 = jax.nn.relu(_gcn_conv(x_B, train_pos_edge_index, W1, b1, dinv))
    x2 = _gcn_conv(x2, train_pos_edge_index, W2, b2, dinv)

    x = jnp.concatenate([x1, x2], axis=1) @ Wm + bm
    total_edge_index = jnp.concatenate([pos_edge_index, neg_edge_index], axis=-1)
    x_j = jnp.take(x, total_edge_index[0], axis=0)
    x_i = jnp.take(x, total_edge_index[1], axis=0)
    att = x2 @ Wa + ba
    res = jnp.einsum('ef,ef->e', x_i, x_j)
    return res, jax.nn.log_softmax(att, axis=1)


# speedup vs baseline: 4.3019x; 3.6254x over previous
import functools

import jax
import jax.numpy as jnp
from jax import lax
from jax.experimental import pallas as pl
from jax.experimental.pallas import tpu as pltpu
from jax.experimental.pallas import tpu_sc as plsc

N = 10000
NPAD = 10112
RPT = NPAD // 16
E = 160000
G = 128
NGRP = E // G
F_IN = 256
H = 128
O = 64
C = 16

NC = 2
NS = 16
NW = NC * NS


def _sc_mesh():
    return plsc.VectorSubcoreMesh(core_axis_name="c", subcore_axis_name="s")


def _zero_vmem2d(buf, rows, cols):
    zero = jnp.zeros((16,), jnp.float32)

    def zb(i, _):
        r = i // (cols // 16)
        c = i % (cols // 16)
        buf[r, pl.ds(c * 16, 16)] = zero
        return 0

    lax.fori_loop(0, rows * (cols // 16), zb, 0)


def _zero_acc_slice(zbuf, acc_sh, sid):
    nfull = RPT // 64
    for t in range(nfull):
        pltpu.sync_copy(zbuf, acc_sh.at[pl.ds(sid * RPT + t * 64, 64)])
    rem = RPT - nfull * 64
    if rem:
        pltpu.sync_copy(zbuf.at[pl.ds(0, rem)],
                        acc_sh.at[pl.ds(sid * RPT + nfull * 64, rem)])


def _sc_degree(dst):
    ept = E // NW
    full = ept // 16
    tail = ept - full * 16

    @functools.partial(
        pl.kernel,
        mesh=_sc_mesh(),
        compiler_params=pltpu.CompilerParams(needs_layout_passes=False),
        out_type=jax.ShapeDtypeStruct((NW, NPAD), jnp.float32),
        scratch_types=[
            pltpu.VMEM((NPAD,), jnp.float32),
            pltpu.VMEM((ept + 16,), jnp.int32),
        ],
    )
    def k(dst_hbm, out_hbm, hist, idx):
        cid = lax.axis_index("c")
        sid = lax.axis_index("s")
        wid = sid * NC + cid

        zero = jnp.zeros((16,), jnp.float32)

        def zbody(i, _):
            hist[pl.ds(i * 16, 16)] = zero
            return 0

        lax.fori_loop(0, NPAD // 16, zbody, 0)

        pltpu.sync_copy(dst_hbm.at[pl.ds(wid * ept, ept)],
                        idx.at[pl.ds(0, ept)])

        ones = jnp.ones((16,), jnp.float32)

        def hbody(i, _):
            v = idx[pl.ds(i * 16, 16)]
            plsc.addupdate_scatter(hist, [v], ones)
            return 0

        lax.fori_loop(0, full, hbody, 0)

        if tail:
            lanes = lax.iota(jnp.int32, 16)
            tmask = lanes < tail
            v = idx[pl.ds(full * 16, 16)]
            v = jnp.where(tmask, v, 0)
            plsc.addupdate_scatter(hist, [v], ones, mask=tmask)

        pltpu.sync_copy(hist, out_hbm.at[wid])

    return k(dst)


K_BUF = 2


def _prop_body(hs, out, srcg_h, dstg_h, si, di, rows, zbuf, acc_sh, gsems,
               sid, n_base, extra, goff):
    _zero_acc_slice(zbuf, acc_sh, sid)
    plsc.subcore_barrier()

    def issue(g, b):
        pltpu.sync_copy(srcg_h.at[g], si.at[b])
        pltpu.sync_copy(dstg_h.at[g], di.at[b])
        return pltpu.async_copy(hs.at[si.at[b]], rows.at[b], gsems[b])

    def scatter(b):
        pltpu.sync_copy(rows.at[b], acc_sh.at[di.at[b]], add=True)

    full_turns = n_base // K_BUF
    rem = n_base - full_turns * K_BUF

    def turn(t, nbuf):
        cps = []
        for b in range(nbuf):
            g = goff + sid + 16 * (t * K_BUF + b)
            cps.append(issue(g, b))
        for owner in range(16):
            @pl.when(sid == owner)
            def _():
                for b in range(nbuf):
                    cps[b].wait()
                    scatter(b)
            plsc.subcore_barrier()

    def turn_loop(t, _):
        turn(t, K_BUF)
        return 0

    lax.fori_loop(0, full_turns, turn_loop, 0)
    if rem:
        turn(full_turns, rem)

    if extra:
        cp = [None]

        @pl.when(sid < extra)
        def _():
            cp[0] = issue(goff + sid + 16 * n_base, 0)

        for owner in range(extra):
            @pl.when(sid == owner)
            def _():
                cp[0].wait()
                scatter(0)
            plsc.subcore_barrier()

    plsc.subcore_barrier()


def _sc_prop1(hsA, hsB, srcg, dstg):
    base_g = NGRP // 16
    extra = NGRP - base_g * 16

    @functools.partial(
        pl.kernel,
        mesh=_sc_mesh(),
        compiler_params=pltpu.CompilerParams(needs_layout_passes=False),
        out_type=(jax.ShapeDtypeStruct((NPAD, H), jnp.float32),
                  jax.ShapeDtypeStruct((NPAD, H), jnp.float32)),
        scratch_types=[
            pltpu.VMEM((K_BUF, G), jnp.int32),
            pltpu.VMEM((K_BUF, G), jnp.int32),
            pltpu.VMEM((K_BUF, G, H), jnp.float32),
            pltpu.VMEM((64, H), jnp.float32),
            pltpu.VMEM_SHARED((NPAD, H), jnp.float32),
        ] + [pltpu.SemaphoreType.DMA] * K_BUF,
    )
    def k(hsA_h, hsB_h, srcg_h, dstg_h, outA, outB, si, di, rows, zbuf,
          acc_sh, *gsems):
        cid = lax.axis_index("c")
        sid = lax.axis_index("s")

        _zero_vmem2d(zbuf, 64, H)

        def process(hs, out):
            _prop_body(hs, out, srcg_h, dstg_h, si, di, rows, zbuf, acc_sh,
                       gsems, sid, base_g, extra, 0)
            pltpu.sync_copy(acc_sh.at[pl.ds(sid * RPT, RPT)],
                            out.at[pl.ds(sid * RPT, RPT)])

        @pl.when(cid == 0)
        def _():
            process(hsA_h, outA)

        @pl.when(cid == 1)
        def _():
            process(hsB_h, outB)

    return k(hsA, hsB, srcg, dstg)


def _sc_prop2(hs2, srcg, dstg):
    per_core = NGRP // NC
    base_g = per_core // 16
    extra = per_core - base_g * 16

    @functools.partial(
        pl.kernel,
        mesh=_sc_mesh(),
        compiler_params=pltpu.CompilerParams(needs_layout_passes=False),
        out_type=(jax.ShapeDtypeStruct((NPAD, H), jnp.float32),
                  jax.ShapeDtypeStruct((NPAD, H), jnp.float32)),
        scratch_types=[
            pltpu.VMEM((K_BUF, G), jnp.int32),
            pltpu.VMEM((K_BUF, G), jnp.int32),
            pltpu.VMEM((K_BUF, G, H), jnp.float32),
            pltpu.VMEM((64, H), jnp.float32),
            pltpu.VMEM_SHARED((NPAD, H), jnp.float32),
        ] + [pltpu.SemaphoreType.DMA] * K_BUF,
    )
    def k(hs_h, srcg_h, dstg_h, out0, out1, si, di, rows, zbuf,
          acc_sh, *gsems):
        cid = lax.axis_index("c")
        sid = lax.axis_index("s")

        _zero_vmem2d(zbuf, 64, H)
        _prop_body(hs_h, None, srcg_h, dstg_h, si, di, rows, zbuf, acc_sh,
                   gsems, sid, base_g, extra, cid * per_core)

        @pl.when(cid == 0)
        def _():
            pltpu.sync_copy(acc_sh.at[pl.ds(sid * RPT, RPT)],
                            out0.at[pl.ds(sid * RPT, RPT)])

        @pl.when(cid == 1)
        def _():
            pltpu.sync_copy(acc_sh.at[pl.ds(sid * RPT, RPT)],
                            out1.at[pl.ds(sid * RPT, RPT)])

    return k(hs2, srcg, dstg)


def _sc_edge_dot(x, srcg, dstg):
    e2 = srcg.shape[0] * G
    ngrp = srcg.shape[0]
    base_g = ngrp // NW
    extra = ngrp - base_g * NW

    @functools.partial(
        pl.kernel,
        mesh=_sc_mesh(),
        compiler_params=pltpu.CompilerParams(needs_layout_passes=False,
                                             use_tc_tiling_on_sc=False),
        out_type=jax.ShapeDtypeStruct((e2,), jnp.float32),
        scratch_types=[
            pltpu.VMEM((G,), jnp.int32),
            pltpu.VMEM((G,), jnp.int32),
            pltpu.VMEM((G, O), jnp.float32),
            pltpu.VMEM((G, O), jnp.float32),
            pltpu.VMEM((G,), jnp.float32),
            pltpu.SemaphoreType.DMA,
            pltpu.SemaphoreType.DMA,
        ],
    )
    def k(x_h, srcg_h, dstg_h, res, si, di, xs, xd, out_v, sem0, sem1):
        cid = lax.axis_index("c")
        sid = lax.axis_index("s")
        wid = sid * NC + cid

        zero = jnp.zeros((16,), jnp.float32)

        def do_group(g):
            pltpu.sync_copy(srcg_h.at[g], si)
            pltpu.sync_copy(dstg_h.at[g], di)
            cp0 = pltpu.async_copy(x_h.at[si], xs, sem0)
            cp1 = pltpu.async_copy(x_h.at[di], xd, sem1)
            cp0.wait()
            cp1.wait()

            for z in range(G // 16):
                out_v[pl.ds(z * 16, 16)] = zero

            def edge(e, _):
                p = xs[e, pl.ds(0, 16)] * xd[e, pl.ds(0, 16)]
                for j in range(1, O // 16):
                    p = p + xs[e, pl.ds(j * 16, 16)] * xd[e, pl.ds(j * 16, 16)]
                eidx = jnp.broadcast_to(e, (16,)).astype(jnp.int32)
                plsc.addupdate_scatter(out_v, [eidx], p)
                return 0

            lax.fori_loop(0, G, edge, 0)
            pltpu.sync_copy(out_v, res.at[pl.ds(g * G, G)])

        def body(k_, _):
            do_group(wid + NW * k_)
            return 0

        lax.fori_loop(0, base_g, body, 0)

        @pl.when(wid < extra)
        def _():
            do_group(wid + NW * base_g)

    return k(x, srcg, dstg)


def _tc_dinv(part):
    part3 = part.reshape(NW, NPAD // 128, 128)

    def body(p_ref, o_ref):
        deg = jnp.sum(p_ref[...], axis=0) + 1.0
        o_ref[...] = lax.rsqrt(deg)

    out = pl.pallas_call(
        body,
        out_shape=jax.ShapeDtypeStruct((NPAD // 128, 128), jnp.float32),
    )(part3)
    return out.reshape(NPAD)


def _tc_mm_scale(x, w, dinv2, block_rows=2000):
    m, kdim = x.shape
    n = w.shape[1]

    def body(x_ref, w_ref, d_ref, o_ref):
        o_ref[...] = jnp.dot(x_ref[...], w_ref[...],
                             preferred_element_type=jnp.float32) * d_ref[...]

    return pl.pallas_call(
        body,
        grid=(m // block_rows,),
        in_specs=[
            pl.BlockSpec((block_rows, kdim), lambda i: (i, 0)),
            pl.BlockSpec((kdim, n), lambda i: (0, 0)),
            pl.BlockSpec((block_rows, 1), lambda i: (i, 0)),
        ],
        out_specs=pl.BlockSpec((block_rows, n), lambda i: (i, 0)),
        out_shape=jax.ShapeDtypeStruct((m, n), jnp.float32),
    )(x, w, dinv2)


def _tc_layer2(accA, hsA, accB, hsB, dinv2, b1, W2, block_rows=2000):
    m = accA.shape[0]

    def body(aA, hA, aB, hB, d_ref, b_ref, w_ref, o_ref):
        d = d_ref[...]
        tA = jax.nn.relu(d * (aA[...] + hA[...]) + b_ref[...])
        tB = jax.nn.relu(d * (aB[...] + hB[...]) + b_ref[...])
        oA = jnp.dot(tA, w_ref[...], preferred_element_type=jnp.float32) * d
        oB = jnp.dot(tB, w_ref[...], preferred_element_type=jnp.float32) * d
        o_ref[...] = jnp.concatenate([oA, oB], axis=1)

    return pl.pallas_call(
        body,
        grid=(m // block_rows,),
        in_specs=[
            pl.BlockSpec((block_rows, H), lambda i: (i, 0)),
            pl.BlockSpec((block_rows, H), lambda i: (i, 0)),
            pl.BlockSpec((block_rows, H), lambda i: (i, 0)),
            pl.BlockSpec((block_rows, H), lambda i: (i, 0)),
            pl.BlockSpec((block_rows, 1), lambda i: (i, 0)),
            pl.BlockSpec((1, H), lambda i: (0, 0)),
            pl.BlockSpec((H, O), lambda i: (0, 0)),
        ],
        out_specs=pl.BlockSpec((block_rows, 2 * O), lambda i: (i, 0)),
        out_shape=jax.ShapeDtypeStruct((m, 2 * O), jnp.float32),
    )(accA, hsA, accB, hsB, dinv2, b1.reshape(1, H), W2)


def _tc_final(acc0, acc1, hs2, dinv2, b2, Wm, bm, Wa, ba, block_rows=2000):
    m = hs2.shape[0]

    def body(a0, a1, h_ref, d_ref, b2_ref, wm_ref, bm_ref, wa_ref, ba_ref,
             x_ref, att_ref):
        d = d_ref[...]
        xc = d * (a0[...] + a1[...] + h_ref[...]) + b2_ref[...]
        x_ref[...] = jnp.dot(xc, wm_ref[...],
                             preferred_element_type=jnp.float32) + bm_ref[...]
        x2 = xc[:, O:]
        t = jnp.dot(x2, wa_ref[...],
                    preferred_element_type=jnp.float32) + ba_ref[...]
        tm = jnp.max(t, axis=1, keepdims=True)
        tt = t - tm
        att_ref[...] = tt - jnp.log(jnp.sum(jnp.exp(tt), axis=1,
                                            keepdims=True))

    b22 = jnp.concatenate([b2, b2]).reshape(1, 2 * O)
    return pl.pallas_call(
        body,
        grid=(m // block_rows,),
        in_specs=[
            pl.BlockSpec((block_rows, 2 * O), lambda i: (i, 0)),
            pl.BlockSpec((block_rows, 2 * O), lambda i: (i, 0)),
            pl.BlockSpec((block_rows, 2 * O), lambda i: (i, 0)),
            pl.BlockSpec((block_rows, 1), lambda i: (i, 0)),
            pl.BlockSpec((1, 2 * O), lambda i: (0, 0)),
            pl.BlockSpec((2 * O, O), lambda i: (0, 0)),
            pl.BlockSpec((1, O), lambda i: (0, 0)),
            pl.BlockSpec((O, C), lambda i: (0, 0)),
            pl.BlockSpec((1, C), lambda i: (0, 0)),
        ],
        out_specs=[
            pl.BlockSpec((block_rows, O), lambda i: (i, 0)),
            pl.BlockSpec((block_rows, C), lambda i: (i, 0)),
        ],
        out_shape=[
            jax.ShapeDtypeStruct((m, O), jnp.float32),
            jax.ShapeDtypeStruct((m, C), jnp.float32),
        ],
    )(acc0, acc1, hs2, dinv2, b22, Wm, bm.reshape(1, O), Wa,
      ba.reshape(1, C))


def kernel(x_A, x_B, train_pos_edge_index, pos_edge_index, neg_edge_index,
           W1, b1, W2, b2, Wm, bm, Wa, ba):
    src = train_pos_edge_index[0]
    dst = train_pos_edge_index[1]
    srcg = src.reshape(NGRP, G)
    dstg = dst.reshape(NGRP, G)

    part = _sc_degree(dst)
    dinv = _tc_dinv(part)[:N]
    dinv2 = dinv[:, None]

    hsA = _tc_mm_scale(x_A, W1, dinv2)
    hsB = _tc_mm_scale(x_B, W1, dinv2)

    accA, accB = _sc_prop1(hsA, hsB, srcg, dstg)

    hs2 = _tc_layer2(accA[:N], hsA, accB[:N], hsB, dinv2, b1, W2)

    acc0, acc1 = _sc_prop2(hs2, srcg, dstg)

    x, att = _tc_final(acc0[:N], acc1[:N], hs2, dinv2, b2, Wm, bm, Wa, ba)

    tot = jnp.concatenate([pos_edge_index, neg_edge_index], axis=-1)
    e2 = tot.shape[1]
    res = _sc_edge_dot(x, tot[0].reshape(e2 // G, G), tot[1].reshape(e2 // G, G))
    return res, att
